# baseline (speedup 1.0000x reference)
import functools

import jax
import jax.numpy as jnp
from jax import lax
from jax.experimental import pallas as pl
from jax.experimental.pallas import tpu as pltpu
from jax.experimental.pallas import tpu_sc as plsc


@functools.lru_cache(maxsize=None)
def _make_gather(B, VU, VP, D):
    info = plsc.get_sparse_core_info()
    NC, NS = info.num_cores, info.num_subcores
    NW = NC * NS
    assert B % NW == 0
    b_per_w = B // NW

    mesh = plsc.VectorSubcoreMesh(core_axis_name="c", subcore_axis_name="s")

    @functools.partial(
        pl.kernel,
        mesh=mesh,
        compiler_params=pltpu.CompilerParams(use_tc_tiling_on_sc=False),
        out_type=(
            jax.ShapeDtypeStruct((B, D), jnp.float32),
            jax.ShapeDtypeStruct((B, D), jnp.float32),
        ),
        scratch_types=[
            pltpu.VMEM((b_per_w,), jnp.int32),
            pltpu.VMEM((b_per_w,), jnp.int32),
            pltpu.VMEM((b_per_w, D), jnp.float32),
            pltpu.VMEM((b_per_w, D), jnp.float32),
            pltpu.SemaphoreType.DMA,
            pltpu.SemaphoreType.DMA,
        ],
    )
    def gather(uid_hbm, pid_hbm, utab_hbm, ptab_hbm, uout_hbm, pout_hbm,
               uidx_v, pidx_v, urows_v, prows_v, usem, psem):
        wid = lax.axis_index("s") * NC + lax.axis_index("c")
        base = wid * b_per_w
        pltpu.sync_copy(uid_hbm.at[pl.ds(base, b_per_w)], uidx_v)
        pltpu.sync_copy(pid_hbm.at[pl.ds(base, b_per_w)], pidx_v)
        ucp = pltpu.async_copy(utab_hbm.at[uidx_v], urows_v, usem)
        pcp = pltpu.async_copy(ptab_hbm.at[pidx_v], prows_v, psem)
        ucp.wait()
        pltpu.sync_copy(urows_v, uout_hbm.at[pl.ds(base, b_per_w)])
        pcp.wait()
        pltpu.sync_copy(prows_v, pout_hbm.at[pl.ds(base, b_per_w)])

    return gather


def _mlp_body(u, p, w1u, w1p, b1, w2, b2, w3, b3, out):
    h1 = jnp.dot(u[...], w1u[...], preferred_element_type=jnp.float32)
    h1 += jnp.dot(p[...], w1p[...], preferred_element_type=jnp.float32)
    h1 = jnp.maximum(h1 + b1[...], 0.0)
    h2 = jnp.maximum(
        jnp.dot(h1, w2[...], preferred_element_type=jnp.float32) + b2[...], 0.0)
    out[...] = jnp.dot(h2, w3[...], preferred_element_type=jnp.float32) + b3[...]


@functools.lru_cache(maxsize=None)
def _make_mlp(B, D, H1, H2, BLK):
    grid = B // BLK
    full = lambda i: (0, 0)
    return pl.pallas_call(
        _mlp_body,
        grid=(grid,),
        in_specs=[
            pl.BlockSpec((BLK, D), lambda i: (i, 0)),
            pl.BlockSpec((BLK, D), lambda i: (i, 0)),
            pl.BlockSpec((D, H1), full),
            pl.BlockSpec((D, H1), full),
            pl.BlockSpec((1, H1), full),
            pl.BlockSpec((H1, H2), full),
            pl.BlockSpec((1, H2), full),
            pl.BlockSpec((H2, 1), full),
            pl.BlockSpec((1, 1), full),
        ],
        out_specs=pl.BlockSpec((BLK, 1), lambda i: (i, 0)),
        out_shape=jax.ShapeDtypeStruct((B, 1), jnp.float32),
    )


def kernel(userId, productId, user_table, product_table, W1, b1, W2, b2, W3, b3):
    B = userId.shape[0]
    D = user_table.shape[1]
    H1 = W1.shape[1]
    H2 = W2.shape[1]

    gather = _make_gather(B, user_table.shape[0], product_table.shape[0], D)
    u_emb, p_emb = gather(userId.astype(jnp.int32), productId.astype(jnp.int32),
                          user_table, product_table)

    mlp = _make_mlp(B, D, H1, H2, BLK=2048)
    return mlp(u_emb, p_emb, W1[:D], W1[D:], b1[None, :], W2, b2[None, :],
               W3, b3[None, :])


# speedup vs baseline: 1.4156x; 1.4156x over previous
import functools

import jax
import jax.numpy as jnp
from jax import lax
from jax.experimental import pallas as pl
from jax.experimental.pallas import tpu as pltpu
from jax.experimental.pallas import tpu_sc as plsc


_CHUNK = 16


@functools.lru_cache(maxsize=None)
def _make_gather(B, D):
    info = plsc.get_sparse_core_info()
    NC, NS = info.num_cores, info.num_subcores
    NW = NC * NS
    assert B % NW == 0
    b_per_w = B // NW
    assert b_per_w % _CHUNK == 0

    mesh = plsc.VectorSubcoreMesh(core_axis_name="c", subcore_axis_name="s")

    @functools.partial(
        pl.kernel,
        mesh=mesh,
        out_type=(
            jax.ShapeDtypeStruct((B, D), jnp.float32),
            jax.ShapeDtypeStruct((B, D), jnp.float32),
        ),
        scratch_types=[
            pltpu.VMEM((b_per_w,), jnp.int32),
            pltpu.VMEM((b_per_w, D), jnp.float32),
            pltpu.SemaphoreType.DMA,
        ],
    )
    def gather(uid_hbm, pid_hbm, utab_hbm, ptab_hbm, uout_hbm, pout_hbm,
               idx_s, rows_v, sem):
        wid = lax.axis_index("s") * NC + lax.axis_index("c")
        base = wid * b_per_w

        def one_table(id_hbm, tab_hbm, out_hbm):
            pltpu.sync_copy(id_hbm.at[pl.ds(base, b_per_w)], idx_s)

            def chunk(c, _):
                off = c * _CHUNK
                idx16 = idx_s[pl.ds(off, _CHUNK)]
                cps = [
                    pltpu.async_copy(
                        tab_hbm.at[pl.ds(idx16[j], 1)],
                        rows_v.at[pl.ds(off + j, 1)], sem)
                    for j in range(_CHUNK)
                ]
                for cp in cps:
                    cp.wait()
                return ()

            lax.fori_loop(0, b_per_w // _CHUNK, chunk, ())
            pltpu.sync_copy(rows_v, out_hbm.at[pl.ds(base, b_per_w)])

        one_table(uid_hbm, utab_hbm, uout_hbm)
        one_table(pid_hbm, ptab_hbm, pout_hbm)

    return gather


def _mlp_body(u, p, w1u, w1p, b1, w2, b2, w3, b3, out):
    h1 = jnp.dot(u[...], w1u[...], preferred_element_type=jnp.float32)
    h1 += jnp.dot(p[...], w1p[...], preferred_element_type=jnp.float32)
    h1 = jnp.maximum(h1 + b1[...], 0.0)
    h2 = jnp.maximum(
        jnp.dot(h1, w2[...], preferred_element_type=jnp.float32) + b2[...], 0.0)
    out[...] = jnp.dot(h2, w3[...], preferred_element_type=jnp.float32) + b3[...]


@functools.lru_cache(maxsize=None)
def _make_mlp(B, D, H1, H2, BLK):
    grid = B // BLK
    full = lambda i: (0, 0)
    return pl.pallas_call(
        _mlp_body,
        grid=(grid,),
        in_specs=[
            pl.BlockSpec((BLK, D), lambda i: (i, 0)),
            pl.BlockSpec((BLK, D), lambda i: (i, 0)),
            pl.BlockSpec((D, H1), full),
            pl.BlockSpec((D, H1), full),
            pl.BlockSpec((1, H1), full),
            pl.BlockSpec((H1, H2), full),
            pl.BlockSpec((1, H2), full),
            pl.BlockSpec((H2, 1), full),
            pl.BlockSpec((1, 1), full),
        ],
        out_specs=pl.BlockSpec((BLK, 1), lambda i: (i, 0)),
        out_shape=jax.ShapeDtypeStruct((B, 1), jnp.float32),
    )


def kernel(userId, productId, user_table, product_table, W1, b1, W2, b2, W3, b3):
    B = userId.shape[0]
    D = user_table.shape[1]
    H1 = W1.shape[1]
    H2 = W2.shape[1]

    gather = _make_gather(B, D)
    u_emb, p_emb = gather(userId.astype(jnp.int32), productId.astype(jnp.int32),
                          user_table, product_table)

    mlp = _make_mlp(B, D, H1, H2, BLK=2048)
    return mlp(u_emb, p_emb, W1[:D], W1[D:], b1[None, :], W2, b2[None, :],
               W3, b3[None, :])
